# baseline (speedup 1.0000x reference)
import functools

import jax
import jax.numpy as jnp
from jax import lax
from jax.experimental import pallas as pl
from jax.experimental.pallas import tpu as pltpu
from jax.experimental.pallas import tpu_sc as plsc

N = 50000
E = 800000
NP = 50176
D = 8
NC, NS = 2, 16
NW = NC * NS
CH = 125
CPT = 200
RPT = NP // NS
NBUF = 8
F32 = jnp.float32

_MESH = plsc.VectorSubcoreMesh(core_axis_name="c", subcore_axis_name="s",
                               num_cores=NC, num_subcores=NS)
_SC_PARAMS = pltpu.CompilerParams(needs_layout_passes=False,
                                  use_tc_tiling_on_sc=False)


def _zero_acc(zbuf, acc, base):
    pltpu.sync_copy(zbuf, acc.at[pl.ds(base, RPT)])


def _scatter_ones(idx, ones_t, acc, ssems):
    nb = len(ssems)
    def group(g, _):
        for b in range(nb):
            j = g * nb + b
            pltpu.async_copy(ones_t, acc.at[idx.at[j]], ssems[b], add=True)
        for b in range(nb):
            j = g * nb + b
            pltpu.make_async_copy(ones_t, acc.at[idx.at[j]], ssems[b]).wait()
        return 0
    lax.fori_loop(0, CPT // nb, group, 0)


def _gather_scatter(h_hbm, isrc, idst, acc, mbufs, gsems, ssems):
    for b in range(NBUF):
        pltpu.async_copy(h_hbm.at[isrc.at[b]], mbufs[b], gsems[b])

    def group(g, _):
        for b in range(NBUF):
            j = g * NBUF + b
            pltpu.make_async_copy(h_hbm.at[isrc.at[j]], mbufs[b],
                                  gsems[b]).wait()
            pltpu.async_copy(mbufs[b], acc.at[idst.at[j]], ssems[b], add=True)
        for b in range(NBUF):
            j = g * NBUF + b
            pltpu.make_async_copy(mbufs[b], acc.at[idst.at[j]],
                                  ssems[b]).wait()
            pltpu.async_copy(h_hbm.at[isrc.at[j + NBUF]], mbufs[b], gsems[b])
        return 0
    lax.fori_loop(0, CPT // NBUF - 1, group, 0)
    for b in range(NBUF):
        j = CPT - NBUF + b
        pltpu.make_async_copy(h_hbm.at[isrc.at[j]], mbufs[b], gsems[b]).wait()
        pltpu.async_copy(mbufs[b], acc.at[idst.at[j]], ssems[b], add=True)
    for b in range(NBUF):
        j = CPT - NBUF + b
        pltpu.make_async_copy(mbufs[b], acc.at[idst.at[j]], ssems[b]).wait()


def _make_agg(n_passes):
    @functools.partial(
        pl.kernel,
        out_type=[jax.ShapeDtypeStruct((NC, NP, D), F32)] * n_passes,
        mesh=_MESH,
        compiler_params=_SC_PARAMS,
        scratch_types=[
            pltpu.VMEM((CPT, CH), jnp.int32),
            pltpu.VMEM((CPT, CH), jnp.int32),
            pltpu.VMEM((RPT, D), F32),
        ] + [pltpu.VMEM((CH, D), F32)] * NBUF
          + [pltpu.SemaphoreType.DMA] * (2 * NBUF)
          + [pltpu.VMEM_SHARED((NP, D), F32)],
    )
    def agg(*refs):
        hs = refs[:n_passes]
        ei = refs[n_passes]
        zeros_h = refs[n_passes + 1]
        outs = refs[n_passes + 2:2 * n_passes + 2]
        k = 2 * n_passes + 2
        isrc, idst, zbuf = refs[k:k + 3]
        mbufs = refs[k + 3:k + 3 + NBUF]
        gsems = refs[k + 3 + NBUF:k + 3 + 2 * NBUF]
        ssems = refs[k + 3 + 2 * NBUF:k + 3 + 3 * NBUF]
        acc = refs[k + 3 + 3 * NBUF]

        c = lax.axis_index("c")
        s = lax.axis_index("s")
        w = c * NS + s
        base = s * RPT
        pltpu.sync_copy(ei.at[0, w], isrc)
        pltpu.sync_copy(ei.at[1, w], idst)
        pltpu.sync_copy(zeros_h, zbuf)

        for p in range(n_passes):
            _zero_acc(zbuf, acc, base)
            plsc.subcore_barrier()
            _gather_scatter(hs[p], isrc, idst, acc, mbufs, gsems, ssems)
            plsc.subcore_barrier()
            pltpu.sync_copy(acc.at[pl.ds(base, RPT)],
                            outs[p].at[c, pl.ds(base, RPT)])
            plsc.subcore_barrier()

    return agg


@functools.partial(
    pl.kernel,
    out_type=[jax.ShapeDtypeStruct((NC, NP, D), F32)] * 2,
    mesh=_MESH,
    compiler_params=_SC_PARAMS,
    scratch_types=[
        pltpu.VMEM((CPT, CH), jnp.int32),
        pltpu.VMEM((CPT, CH), jnp.int32),
        pltpu.VMEM((RPT, D), F32),
        pltpu.VMEM((CH, D), F32),
    ] + [pltpu.SemaphoreType.DMA] * 4
      + [pltpu.VMEM_SHARED((NP, D), F32)],
)
def _deg(ei, zeros_h, ones_h, out_src, out_dst,
         isrc, idst, zbuf, ones_t, s0, s1, s2, s3, acc):
    ssems = (s0, s1, s2, s3)
    c = lax.axis_index("c")
    s = lax.axis_index("s")
    w = c * NS + s
    base = s * RPT
    pltpu.sync_copy(ei.at[0, w], isrc)
    pltpu.sync_copy(ei.at[1, w], idst)
    pltpu.sync_copy(zeros_h, zbuf)
    pltpu.sync_copy(ones_h, ones_t)

    for idx, out in ((isrc, out_src), (idst, out_dst)):
        _zero_acc(zbuf, acc, base)
        plsc.subcore_barrier()
        _scatter_ones(idx, ones_t, acc, ssems)
        plsc.subcore_barrier()
        pltpu.sync_copy(acc.at[pl.ds(base, RPT)], out.at[c, pl.ds(base, RPT)])
        plsc.subcore_barrier()


_agg1 = _make_agg(1)
_agg2 = _make_agg(2)
_agg8 = _make_agg(8)


NT = NP // 16
_B = 112


def _bd(W):
    P, Q = W.shape[0] // 8, W.shape[1] // 8
    i = jnp.arange(128 * P)
    j = jnp.arange(128 * Q)
    ri = 8 * (i // 128) + i % 8
    cj = 8 * (j // 128) + j % 8
    R = (ri[:, None] == jnp.arange(8 * P)[None, :]).astype(F32)
    C = (cj[:, None] == jnp.arange(8 * Q)[None, :]).astype(F32)
    mask = ((i[:, None] // 8) % 16 == (j[None, :] // 8) % 16).astype(F32)
    spread = jnp.dot(jnp.dot(R, W), C.T, precision=jax.lax.Precision.HIGHEST)
    return spread * mask


def _bt(b):
    Q = b.shape[0] // 8
    return jnp.broadcast_to(b.reshape(Q, 1, 8), (Q, 16, 8)).reshape(1, Q * 128)


def _norms_t(do_blk, di_blk, i):
    deg_out = do_blk[0] + do_blk[1]
    deg_in = di_blk[0] + di_blk[1]
    r = lax.broadcasted_iota(jnp.int32, (_B, 128), 0)
    l = lax.broadcasted_iota(jnp.int32, (_B, 128), 1)
    live = 16 * (i * _B + r) + l // 8 < N
    nsrc = jnp.where(jnp.logical_and(live, deg_out > 0),
                     lax.rsqrt(jnp.maximum(deg_out, 1.0)), 0.0)
    ndst = jnp.where(jnp.logical_and(live, deg_in > 0),
                     lax.rsqrt(jnp.maximum(deg_in, 1.0)), 0.0)
    return nsrc, ndst


_ROW_SPEC = pl.BlockSpec((_B, 128), lambda i: (i, 0))
_DEG_SPEC = pl.BlockSpec((NC, _B, 128), lambda i: (0, i, 0))


def _t(x):
    return x.reshape(x.shape[:-2] + (NT, 128))


def _prep_body(fp_ref, do_ref, di_ref, h1_ref):
    nsrc, _ = _norms_t(do_ref[...], di_ref[...], pl.program_id(0))
    h1_ref[...] = fp_ref[...] * nsrc


def _prep(fp, pdo, pdi):
    out = pl.pallas_call(
        _prep_body,
        grid=(NT // _B,),
        in_specs=[_ROW_SPEC, _DEG_SPEC, _DEG_SPEC],
        out_specs=_ROW_SPEC,
        out_shape=jax.ShapeDtypeStruct((NT, 128), F32),
    )(_t(fp), _t(pdo), _t(pdi))
    return out.reshape(NP, D)


def _layer1_body(p_ref, do_ref, di_ref, w_ref, b_ref, *out_refs):
    nsrc, ndst = _norms_t(do_ref[...], di_ref[...], pl.program_id(0))
    agg = (p_ref[0] + p_ref[1]) * ndst
    x = jnp.maximum(jnp.dot(agg, w_ref[...],
                            preferred_element_type=F32) + b_ref[...], 0.0)
    for p in range(8):
        out_refs[p][...] = x[:, p * 128:(p + 1) * 128] * nsrc


def _layer1(p1, pdo, pdi, BD1, b1_t):
    outs = pl.pallas_call(
        _layer1_body,
        grid=(NT // _B,),
        in_specs=[
            _DEG_SPEC, _DEG_SPEC, _DEG_SPEC,
            pl.BlockSpec((128, 1024), lambda i: (0, 0)),
            pl.BlockSpec((1, 1024), lambda i: (0, 0)),
        ],
        out_specs=[_ROW_SPEC] * 8,
        out_shape=[jax.ShapeDtypeStruct((NT, 128), F32)] * 8,
    )(_t(p1), _t(pdo), _t(pdi), BD1, b1_t)
    return [o.reshape(NP, D) for o in outs]


def _layer2_body(*refs):
    ps = refs[:8]
    do_ref, di_ref, w2_ref, b2_ref, w3_ref, ga_ref, gb_ref = refs[8:]
    nsrc, ndst = _norms_t(do_ref[...], di_ref[...], pl.program_id(0))
    agg = jnp.concatenate([(p[0] + p[1]) * ndst for p in ps], axis=1)
    x2 = jnp.maximum(jnp.dot(agg, w2_ref[...],
                             preferred_element_type=F32) + b2_ref[...], 0.0)
    x2 = x2 * jnp.concatenate([nsrc] * 8, axis=1)
    g3 = jnp.dot(x2, w3_ref[...], preferred_element_type=F32)
    ga_ref[...] = g3[:, :128]
    gb_ref[...] = g3[:, 128:]


def _layer2(pXs, pdo, pdi, BDW2, b2_t, BDW3):
    ga, gb = pl.pallas_call(
        _layer2_body,
        grid=(NT // _B,),
        in_specs=[_DEG_SPEC] * 8 + [
            _DEG_SPEC, _DEG_SPEC,
            pl.BlockSpec((1024, 1024), lambda i: (0, 0)),
            pl.BlockSpec((1, 1024), lambda i: (0, 0)),
            pl.BlockSpec((1024, 256), lambda i: (0, 0)),
        ],
        out_specs=[_ROW_SPEC] * 2,
        out_shape=[jax.ShapeDtypeStruct((NT, 128), F32)] * 2,
    )(*[_t(p) for p in pXs], _t(pdo), _t(pdi), BDW2, b2_t, BDW3)
    return ga.reshape(NP, D), gb.reshape(NP, D)


def _final_body(pa_ref, pb_ref, do_ref, di_ref, b_ref, out_ref):
    _, ndst = _norms_t(do_ref[...], di_ref[...], pl.program_id(0))
    za = (pa_ref[0] + pa_ref[1]) * ndst + b_ref[:, :128]
    zb = (pb_ref[0] + pb_ref[1]) * ndst + b_ref[:, 128:]
    za3 = za.reshape(_B, 16, 8)
    zb3 = zb.reshape(_B, 16, 8)
    m = jnp.maximum(jnp.max(za3, axis=2, keepdims=True),
                    jnp.max(zb3, axis=2, keepdims=True))
    ea = jnp.exp(za3 - m)
    eb = jnp.exp(zb3 - m)
    s = jnp.sum(ea, axis=2, keepdims=True) + jnp.sum(eb, axis=2, keepdims=True)
    out_ref[...] = jnp.concatenate([ea / s, eb / s],
                                   axis=2).reshape(16 * _B, 16)


def _final(p3a, p3b, pdo, pdi, b3_t):
    return pl.pallas_call(
        _final_body,
        grid=(NT // _B,),
        in_specs=[
            _DEG_SPEC, _DEG_SPEC, _DEG_SPEC, _DEG_SPEC,
            pl.BlockSpec((1, 256), lambda i: (0, 0)),
        ],
        out_specs=pl.BlockSpec((16 * _B, 16), lambda i: (i, 0)),
        out_shape=jax.ShapeDtypeStruct((N, 16), F32),
    )(_t(p3a), _t(p3b), _t(pdo), _t(pdi), b3_t)


def kernel(features, edge_index, W1, b1, W2, b2, W3, b3):
    ei = edge_index.astype(jnp.int32).reshape(2, NW, CPT, CH)

    zeros_h = jnp.zeros((RPT, D), F32)
    ones_h = jnp.ones((CH, D), F32)
    fp = jnp.zeros((NP, D), F32).at[:N, :7].set(features)
    W1p = jnp.zeros((D, 64), F32).at[:7].set(W1)
    BD1, BDW2, BDW3 = _bd(W1p), _bd(W2), _bd(W3)
    b1_t, b2_t, b3_t = _bt(b1), _bt(b2), _bt(b3)

    pdo, pdi = _deg(ei, zeros_h, ones_h)
    h1 = _prep(fp, pdo, pdi)
    (p1,) = _agg1(h1, ei, zeros_h)
    h2s = _layer1(p1, pdo, pdi, BD1, b1_t)
    pXs = _agg8(*h2s, ei, zeros_h)
    g3a, g3b = _layer2(pXs, pdo, pdi, BDW2, b2_t, BDW3)
    p3a, p3b = _agg2(g3a, g3b, ei, zeros_h)
    return _final(p3a, p3b, pdo, pdi, b3_t)


# speedup vs baseline: 1.1256x; 1.1256x over previous
import functools

import jax
import jax.numpy as jnp
from jax import lax
from jax.experimental import pallas as pl
from jax.experimental.pallas import tpu as pltpu
from jax.experimental.pallas import tpu_sc as plsc

N = 50000
E = 800000
NP = 50176
D = 8
NC, NS = 2, 16
NW = NC * NS
CH = 125
CPT = 200
RPT = NP // NS
NBUF = 8
F32 = jnp.float32

_MESH = plsc.VectorSubcoreMesh(core_axis_name="c", subcore_axis_name="s",
                               num_cores=NC, num_subcores=NS)
_SC_PARAMS = pltpu.CompilerParams(needs_layout_passes=False,
                                  use_tc_tiling_on_sc=False)


def _zero_acc(zbuf, acc, base):
    pltpu.sync_copy(zbuf, acc.at[pl.ds(base, RPT)])


def _scatter_ones(idx, ones_t, acc, ssems):
    nb = len(ssems)
    def group(g, _):
        for b in range(nb):
            j = g * nb + b
            pltpu.async_copy(ones_t, acc.at[idx.at[j]], ssems[b], add=True)
        for b in range(nb):
            j = g * nb + b
            pltpu.make_async_copy(ones_t, acc.at[idx.at[j]], ssems[b]).wait()
        return 0
    lax.fori_loop(0, CPT // nb, group, 0)


def _gather_scatter(h_hbm, isrc, idst, acc, mbufs, gsems, ssems):
    for b in range(NBUF):
        pltpu.async_copy(h_hbm.at[isrc.at[b]], mbufs[b], gsems[b])

    def group(g, _):
        for b in range(NBUF):
            j = g * NBUF + b
            pltpu.make_async_copy(h_hbm.at[isrc.at[j]], mbufs[b],
                                  gsems[b]).wait()
            pltpu.async_copy(mbufs[b], acc.at[idst.at[j]], ssems[b], add=True)
            pltpu.make_async_copy(mbufs[b], acc.at[idst.at[j]],
                                  ssems[b]).wait()
            pltpu.async_copy(h_hbm.at[isrc.at[j + NBUF]], mbufs[b], gsems[b])
        return 0
    lax.fori_loop(0, CPT // NBUF - 1, group, 0)
    for b in range(NBUF):
        j = CPT - NBUF + b
        pltpu.make_async_copy(h_hbm.at[isrc.at[j]], mbufs[b], gsems[b]).wait()
        pltpu.async_copy(mbufs[b], acc.at[idst.at[j]], ssems[b], add=True)
        pltpu.make_async_copy(mbufs[b], acc.at[idst.at[j]], ssems[b]).wait()


def _make_agg(n_passes):
    @functools.partial(
        pl.kernel,
        out_type=[jax.ShapeDtypeStruct((NC, NP, D), F32)] * n_passes,
        mesh=_MESH,
        compiler_params=_SC_PARAMS,
        scratch_types=[
            pltpu.VMEM((CPT, CH), jnp.int32),
            pltpu.VMEM((CPT, CH), jnp.int32),
            pltpu.VMEM((RPT, D), F32),
        ] + [pltpu.VMEM((CH, D), F32)] * NBUF
          + [pltpu.SemaphoreType.DMA] * (2 * NBUF)
          + [pltpu.VMEM_SHARED((NP, D), F32)],
    )
    def agg(*refs):
        hs = refs[:n_passes]
        ei = refs[n_passes]
        zeros_h = refs[n_passes + 1]
        outs = refs[n_passes + 2:2 * n_passes + 2]
        k = 2 * n_passes + 2
        isrc, idst, zbuf = refs[k:k + 3]
        mbufs = refs[k + 3:k + 3 + NBUF]
        gsems = refs[k + 3 + NBUF:k + 3 + 2 * NBUF]
        ssems = refs[k + 3 + 2 * NBUF:k + 3 + 3 * NBUF]
        acc = refs[k + 3 + 3 * NBUF]

        c = lax.axis_index("c")
        s = lax.axis_index("s")
        w = c * NS + s
        base = s * RPT
        pltpu.sync_copy(ei.at[0, w], isrc)
        pltpu.sync_copy(ei.at[1, w], idst)
        pltpu.sync_copy(zeros_h, zbuf)

        for p in range(n_passes):
            _zero_acc(zbuf, acc, base)
            plsc.subcore_barrier()
            _gather_scatter(hs[p], isrc, idst, acc, mbufs, gsems, ssems)
            plsc.subcore_barrier()
            pltpu.sync_copy(acc.at[pl.ds(base, RPT)],
                            outs[p].at[c, pl.ds(base, RPT)])
            plsc.subcore_barrier()

    return agg


@functools.partial(
    pl.kernel,
    out_type=[jax.ShapeDtypeStruct((NC, NP, D), F32)] * 2,
    mesh=_MESH,
    compiler_params=_SC_PARAMS,
    scratch_types=[
        pltpu.VMEM((CPT, CH), jnp.int32),
        pltpu.VMEM((CPT, CH), jnp.int32),
        pltpu.VMEM((RPT, D), F32),
        pltpu.VMEM((CH, D), F32),
    ] + [pltpu.SemaphoreType.DMA] * 4
      + [pltpu.VMEM_SHARED((NP, D), F32)],
)
def _deg(ei, zeros_h, ones_h, out_src, out_dst,
         isrc, idst, zbuf, ones_t, s0, s1, s2, s3, acc):
    ssems = (s0, s1, s2, s3)
    c = lax.axis_index("c")
    s = lax.axis_index("s")
    w = c * NS + s
    base = s * RPT
    pltpu.sync_copy(ei.at[0, w], isrc)
    pltpu.sync_copy(ei.at[1, w], idst)
    pltpu.sync_copy(zeros_h, zbuf)
    pltpu.sync_copy(ones_h, ones_t)

    for idx, out in ((isrc, out_src), (idst, out_dst)):
        _zero_acc(zbuf, acc, base)
        plsc.subcore_barrier()
        _scatter_ones(idx, ones_t, acc, ssems)
        plsc.subcore_barrier()
        pltpu.sync_copy(acc.at[pl.ds(base, RPT)], out.at[c, pl.ds(base, RPT)])
        plsc.subcore_barrier()


_agg1 = _make_agg(1)
_agg2 = _make_agg(2)
_agg8 = _make_agg(8)


NT = NP // 16
_B = 112


def _bd(W):
    P, Q = W.shape[0] // 8, W.shape[1] // 8
    i = jnp.arange(128 * P)
    j = jnp.arange(128 * Q)
    ri = 8 * (i // 128) + i % 8
    cj = 8 * (j // 128) + j % 8
    R = (ri[:, None] == jnp.arange(8 * P)[None, :]).astype(F32)
    C = (cj[:, None] == jnp.arange(8 * Q)[None, :]).astype(F32)
    mask = ((i[:, None] // 8) % 16 == (j[None, :] // 8) % 16).astype(F32)
    spread = jnp.dot(jnp.dot(R, W), C.T, precision=jax.lax.Precision.HIGHEST)
    return spread * mask


def _bt(b):
    Q = b.shape[0] // 8
    return jnp.broadcast_to(b.reshape(Q, 1, 8), (Q, 16, 8)).reshape(1, Q * 128)


def _norms_t(do_blk, di_blk, i):
    deg_out = do_blk[0] + do_blk[1]
    deg_in = di_blk[0] + di_blk[1]
    r = lax.broadcasted_iota(jnp.int32, (_B, 128), 0)
    l = lax.broadcasted_iota(jnp.int32, (_B, 128), 1)
    live = 16 * (i * _B + r) + l // 8 < N
    nsrc = jnp.where(jnp.logical_and(live, deg_out > 0),
                     lax.rsqrt(jnp.maximum(deg_out, 1.0)), 0.0)
    ndst = jnp.where(jnp.logical_and(live, deg_in > 0),
                     lax.rsqrt(jnp.maximum(deg_in, 1.0)), 0.0)
    return nsrc, ndst


_ROW_SPEC = pl.BlockSpec((_B, 128), lambda i: (i, 0))
_DEG_SPEC = pl.BlockSpec((NC, _B, 128), lambda i: (0, i, 0))


def _t(x):
    return x.reshape(x.shape[:-2] + (NT, 128))


def _prep_body(fp_ref, do_ref, di_ref, h1_ref):
    nsrc, _ = _norms_t(do_ref[...], di_ref[...], pl.program_id(0))
    h1_ref[...] = fp_ref[...] * nsrc


def _prep(fp, pdo, pdi):
    out = pl.pallas_call(
        _prep_body,
        grid=(NT // _B,),
        in_specs=[_ROW_SPEC, _DEG_SPEC, _DEG_SPEC],
        out_specs=_ROW_SPEC,
        out_shape=jax.ShapeDtypeStruct((NT, 128), F32),
    )(_t(fp), _t(pdo), _t(pdi))
    return out.reshape(NP, D)


def _layer1_body(p_ref, do_ref, di_ref, w_ref, b_ref, *out_refs):
    nsrc, ndst = _norms_t(do_ref[...], di_ref[...], pl.program_id(0))
    agg = (p_ref[0] + p_ref[1]) * ndst
    x = jnp.maximum(jnp.dot(agg, w_ref[...],
                            preferred_element_type=F32) + b_ref[...], 0.0)
    for p in range(8):
        out_refs[p][...] = x[:, p * 128:(p + 1) * 128] * nsrc


def _layer1(p1, pdo, pdi, BD1, b1_t):
    outs = pl.pallas_call(
        _layer1_body,
        grid=(NT // _B,),
        in_specs=[
            _DEG_SPEC, _DEG_SPEC, _DEG_SPEC,
            pl.BlockSpec((128, 1024), lambda i: (0, 0)),
            pl.BlockSpec((1, 1024), lambda i: (0, 0)),
        ],
        out_specs=[_ROW_SPEC] * 8,
        out_shape=[jax.ShapeDtypeStruct((NT, 128), F32)] * 8,
    )(_t(p1), _t(pdo), _t(pdi), BD1, b1_t)
    return [o.reshape(NP, D) for o in outs]


def _layer2_body(*refs):
    ps = refs[:8]
    do_ref, di_ref, w2_ref, b2_ref, w3_ref, ga_ref, gb_ref = refs[8:]
    nsrc, ndst = _norms_t(do_ref[...], di_ref[...], pl.program_id(0))
    agg = jnp.concatenate([(p[0] + p[1]) * ndst for p in ps], axis=1)
    x2 = jnp.maximum(jnp.dot(agg, w2_ref[...],
                             preferred_element_type=F32) + b2_ref[...], 0.0)
    x2 = x2 * jnp.concatenate([nsrc] * 8, axis=1)
    g3 = jnp.dot(x2, w3_ref[...], preferred_element_type=F32)
    ga_ref[...] = g3[:, :128]
    gb_ref[...] = g3[:, 128:]


def _layer2(pXs, pdo, pdi, BDW2, b2_t, BDW3):
    ga, gb = pl.pallas_call(
        _layer2_body,
        grid=(NT // _B,),
        in_specs=[_DEG_SPEC] * 8 + [
            _DEG_SPEC, _DEG_SPEC,
            pl.BlockSpec((1024, 1024), lambda i: (0, 0)),
            pl.BlockSpec((1, 1024), lambda i: (0, 0)),
            pl.BlockSpec((1024, 256), lambda i: (0, 0)),
        ],
        out_specs=[_ROW_SPEC] * 2,
        out_shape=[jax.ShapeDtypeStruct((NT, 128), F32)] * 2,
    )(*[_t(p) for p in pXs], _t(pdo), _t(pdi), BDW2, b2_t, BDW3)
    return ga.reshape(NP, D), gb.reshape(NP, D)


def _final_body(pa_ref, pb_ref, do_ref, di_ref, b_ref, out_ref):
    _, ndst = _norms_t(do_ref[...], di_ref[...], pl.program_id(0))
    za = (pa_ref[0] + pa_ref[1]) * ndst + b_ref[:, :128]
    zb = (pb_ref[0] + pb_ref[1]) * ndst + b_ref[:, 128:]
    za3 = za.reshape(_B, 16, 8)
    zb3 = zb.reshape(_B, 16, 8)
    m = jnp.maximum(jnp.max(za3, axis=2, keepdims=True),
                    jnp.max(zb3, axis=2, keepdims=True))
    ea = jnp.exp(za3 - m)
    eb = jnp.exp(zb3 - m)
    s = jnp.sum(ea, axis=2, keepdims=True) + jnp.sum(eb, axis=2, keepdims=True)
    out_ref[...] = jnp.concatenate([ea / s, eb / s],
                                   axis=2).reshape(16 * _B, 16)


def _final(p3a, p3b, pdo, pdi, b3_t):
    return pl.pallas_call(
        _final_body,
        grid=(NT // _B,),
        in_specs=[
            _DEG_SPEC, _DEG_SPEC, _DEG_SPEC, _DEG_SPEC,
            pl.BlockSpec((1, 256), lambda i: (0, 0)),
        ],
        out_specs=pl.BlockSpec((16 * _B, 16), lambda i: (i, 0)),
        out_shape=jax.ShapeDtypeStruct((N, 16), F32),
    )(_t(p3a), _t(p3b), _t(pdo), _t(pdi), b3_t)


def kernel(features, edge_index, W1, b1, W2, b2, W3, b3):
    ei = edge_index.astype(jnp.int32).reshape(2, NW, CPT, CH)

    zeros_h = jnp.zeros((RPT, D), F32)
    ones_h = jnp.ones((CH, D), F32)
    fp = jnp.zeros((NP, D), F32).at[:N, :7].set(features)
    W1p = jnp.zeros((D, 64), F32).at[:7].set(W1)
    BD1, BDW2, BDW3 = _bd(W1p), _bd(W2), _bd(W3)
    b1_t, b2_t, b3_t = _bt(b1), _bt(b2), _bt(b3)

    pdo, pdi = _deg(ei, zeros_h, ones_h)
    h1 = _prep(fp, pdo, pdi)
    (p1,) = _agg1(h1, ei, zeros_h)
    h2s = _layer1(p1, pdo, pdi, BD1, b1_t)
    pXs = _agg8(*h2s, ei, zeros_h)
    g3a, g3b = _layer2(pXs, pdo, pdi, BDW2, b2_t, BDW3)
    p3a, p3b = _agg2(g3a, g3b, ei, zeros_h)
    return _final(p3a, p3b, pdo, pdi, b3_t)
